# baseline (speedup 1.0000x reference)
import functools

import jax
import jax.numpy as jnp
from jax import lax
from jax.experimental import pallas as pl
from jax.experimental.pallas import tpu as pltpu
from jax.experimental.pallas import tpu_sc as plsc

D_MODEL = 64
SCALE = 8.0
NC, NS = 2, 16
NW = NC * NS
CHUNK = 128
ROWS = 4096 * 200
PER_W = ROWS // NW
NCHUNK = PER_W // CHUNK
LANES = 16


def _body(x_hbm, table_hbm, out_hbm, idx_v, buf, sem_g):
    c = lax.axis_index("c")
    s = lax.axis_index("s")
    wid = s * NC + c
    pltpu.sync_copy(x_hbm.at[wid], idx_v)

    def chunk_body(j, _):
        pltpu.async_copy(table_hbm.at[idx_v.at[j]], buf, sem_g).wait()

        def scale_row(i, _):
            for u in range(D_MODEL // LANES):
                sl = pl.ds(u * LANES, LANES)
                buf[i, sl] = buf[i, sl] * SCALE
            return 0

        lax.fori_loop(0, CHUNK, scale_row, 0)
        pltpu.sync_copy(buf, out_hbm.at[wid, j])
        return 0

    lax.fori_loop(0, NCHUNK, chunk_body, 0)


_sc_call = functools.partial(
    pl.kernel,
    out_type=jax.ShapeDtypeStruct((NW, NCHUNK, CHUNK, D_MODEL), jnp.float32),
    mesh=plsc.VectorSubcoreMesh(core_axis_name="c", subcore_axis_name="s"),
    compiler_params=pltpu.CompilerParams(use_tc_tiling_on_sc=False),
    scratch_types=[
        pltpu.VMEM((NCHUNK, CHUNK), jnp.int32),
        pltpu.VMEM((CHUNK, D_MODEL), jnp.float32),
        pltpu.SemaphoreType.DMA,
    ],
)(_body)


@jax.jit
def kernel(x, table):
    xr = x.reshape(NW, NCHUNK, CHUNK)
    out = _sc_call(xr, table)
    return out.reshape(4096, 200, D_MODEL)


# speedup vs baseline: 1.2094x; 1.2094x over previous
import functools

import jax
import jax.numpy as jnp
from jax import lax
from jax.experimental import pallas as pl
from jax.experimental.pallas import tpu as pltpu
from jax.experimental.pallas import tpu_sc as plsc

D_MODEL = 64
SCALE = 8.0
NC, NS = 2, 16
NW = NC * NS
CHUNK = 128
ROWS = 4096 * 200
PER_W = ROWS // NW
NCHUNK = PER_W // CHUNK
LANES = 16
NBUF = 8
AHEAD = 4


def _body(x_hbm, table_hbm, out_hbm, idx_v, *rest):
    bufs = rest[:NBUF]
    sgs = rest[NBUF:2 * NBUF]
    sos = rest[2 * NBUF:3 * NBUF]
    c = lax.axis_index("c")
    s = lax.axis_index("s")
    wid = s * NC + c
    pltpu.sync_copy(x_hbm.at[wid], idx_v)

    def issue_gather(j, b):
        pltpu.async_copy(table_hbm.at[idx_v.at[j]], bufs[b], sgs[b])

    def wait_gather(j, b):
        pltpu.make_async_copy(table_hbm.at[idx_v.at[j]], bufs[b], sgs[b]).wait()

    def issue_out(j, b):
        pltpu.async_copy(bufs[b], out_hbm.at[wid, j], sos[b])

    def wait_out(j, b):
        pltpu.make_async_copy(bufs[b], out_hbm.at[wid, j], sos[b]).wait()

    def scale(b):
        buf = bufs[b]

        def row4(i, _):
            r = i * 4
            for v in range(4):
                for u in range(D_MODEL // LANES):
                    sl = pl.ds(u * LANES, LANES)
                    buf[r + v, sl] = buf[r + v, sl] * SCALE
            return 0

        lax.fori_loop(0, CHUNK // 4, row4, 0)

    def step(j, b, first=False):
        wait_gather(j, b)
        scale(b)
        issue_out(j, b)
        jn = j + AHEAD
        bn = (b + AHEAD) % NBUF
        if not first:
            wait_out(jn - NBUF, bn)
        issue_gather(jn, bn)

    for j in range(AHEAD):
        issue_gather(j, j)
    for b in range(NBUF):
        step(b, b, first=(b < AHEAD))

    def block(g, _):
        j0 = g * NBUF
        for b in range(NBUF):
            step(j0 + b, b)
        return 0

    lax.fori_loop(1, NCHUNK // NBUF - 1, block, 0)

    j0 = NCHUNK - NBUF
    for b in range(AHEAD):
        step(j0 + b, b)
    for b in range(AHEAD, NBUF):
        j = j0 + b
        wait_gather(j, b)
        scale(b)
        issue_out(j, b)
    for b in range(NBUF):
        wait_out(NCHUNK - NBUF + b, b)


_sc_call = functools.partial(
    pl.kernel,
    out_type=jax.ShapeDtypeStruct((NW, NCHUNK, CHUNK, D_MODEL), jnp.float32),
    mesh=plsc.VectorSubcoreMesh(core_axis_name="c", subcore_axis_name="s"),
    compiler_params=pltpu.CompilerParams(use_tc_tiling_on_sc=False),
    scratch_types=(
        [pltpu.VMEM((NCHUNK, CHUNK), jnp.int32)]
        + [pltpu.VMEM((CHUNK, D_MODEL), jnp.float32) for _ in range(NBUF)]
        + [pltpu.SemaphoreType.DMA for _ in range(2 * NBUF)]
    ),
)(_body)


@jax.jit
def kernel(x, table):
    xr = x.reshape(NW, NCHUNK, CHUNK)
    out = _sc_call(xr, table)
    return out.reshape(4096, 200, D_MODEL)
